# Initial kernel scaffold; baseline (speedup 1.0000x reference)
import functools

import jax
import jax.numpy as jnp
from jax import lax
from jax.experimental import pallas as pl
from jax.experimental.pallas import tpu as pltpu
from jax.experimental.pallas import tpu_sc as plsc

D = 64
K = 1024
ROWS = 16 * 1024
BLK = 512
NBLK = ROWS // BLK
BETA = 0.25


def _tc_body(z_ref, se_ref, emb_t_ref, idx_ref, loss_ref):
    i = pl.program_id(0)
    z = z_ref[...]
    sz = jnp.sum(z * z, axis=1, keepdims=True)
    mm = jnp.dot(z, emb_t_ref[...], preferred_element_type=jnp.float32)
    d = (sz + se_ref[...]) - 2.0 * mm
    m = jnp.min(d, axis=1, keepdims=True)
    iota = lax.broadcasted_iota(jnp.int32, (BLK, K), 1)
    idx = jnp.min(jnp.where(d == m, iota, K), axis=1)
    idx_ref[...] = idx.reshape(1, 1, BLK)

    @pl.when(i == 0)
    def _init():
        loss_ref[0, 0] = 0.0

    loss_ref[0, 0] += jnp.sum(m)

    @pl.when(i == NBLK - 1)
    def _finish():
        loss_ref[0, 0] = loss_ref[0, 0] * ((1.0 + BETA) / (ROWS * D))


_tc_call = pl.pallas_call(
    _tc_body,
    grid=(NBLK,),
    in_specs=[
        pl.BlockSpec((BLK, D), lambda i: (i, 0)),
        pl.BlockSpec((1, K), lambda i: (0, 0)),
        pl.BlockSpec((D, K), lambda i: (0, 0)),
    ],
    out_specs=[
        pl.BlockSpec((1, 1, BLK), lambda i: (i, 0, 0)),
        pl.BlockSpec(memory_space=pltpu.SMEM),
    ],
    out_shape=[
        jax.ShapeDtypeStruct((NBLK, 1, BLK), jnp.int32),
        jax.ShapeDtypeStruct((1, 1), jnp.float32),
    ],
)


_info = plsc.get_sparse_core_info()
_NC, _NS = _info.num_cores, _info.num_subcores
_NW = _NC * _NS
BPW = ROWS // _NW
CHUNK = 128
NCH = BPW // CHUNK

_sc_mesh = plsc.VectorSubcoreMesh(core_axis_name="c", subcore_axis_name="s")


@functools.partial(
    pl.kernel,
    out_type=jax.ShapeDtypeStruct((ROWS, D), jnp.float32),
    mesh=_sc_mesh,
    scratch_types=[
        pltpu.VMEM((NCH, CHUNK), jnp.int32),
        pltpu.VMEM((BPW, D), jnp.float32),
        pltpu.SemaphoreType.DMA,
    ],
)
def _sc_gather(emb_hbm, idx_hbm, out_hbm, idx_v, rows_v, sem):
    wid = lax.axis_index("s") * _NC + lax.axis_index("c")
    pltpu.sync_copy(idx_hbm.at[pl.ds(wid * NCH, NCH)], idx_v)
    copies = [
        pltpu.async_copy(
            emb_hbm.at[idx_v.at[j]], rows_v.at[pl.ds(j * CHUNK, CHUNK)], sem
        )
        for j in range(NCH)
    ]
    for c in copies:
        c.wait()
    pltpu.sync_copy(rows_v, out_hbm.at[pl.ds(wid * BPW, BPW)])


def kernel(z, emb_w):
    z_flat = z.reshape(-1, D)
    se = jnp.sum(emb_w ** 2, axis=1).reshape(1, K)
    emb_t = emb_w.T
    idx3d, loss2d = _tc_call(z_flat, se, emb_t)
    idx = idx3d.reshape(ROWS // CHUNK, CHUNK)
    zq = _sc_gather(emb_w, idx)
    return zq.reshape(z.shape), loss2d[0, 0]


# speedup vs baseline: 1.0297x; 1.0297x over previous
import functools

import jax
import jax.numpy as jnp
from jax import lax
from jax.experimental import pallas as pl
from jax.experimental.pallas import tpu as pltpu
from jax.experimental.pallas import tpu_sc as plsc

D = 64
K = 1024
ROWS = 16 * 1024
BLK = 512
NBLK = ROWS // BLK
BETA = 0.25


def _tc_body(z_ref, se_ref, emb_t_ref, idx_ref, loss_ref):
    i = pl.program_id(0)
    z = z_ref[...]
    sz = jnp.sum(z * z, axis=1, keepdims=True)
    mm = jnp.dot(z, emb_t_ref[...], preferred_element_type=jnp.float32)
    d = (sz + se_ref[...]) - 2.0 * mm
    m = jnp.min(d, axis=1, keepdims=True)
    iota = lax.broadcasted_iota(jnp.int32, (BLK, K), 1)
    idx = jnp.min(jnp.where(d == m, iota, K), axis=1)
    idx_ref[...] = idx.reshape(1, 1, BLK)

    @pl.when(i == 0)
    def _init():
        loss_ref[0, 0] = 0.0

    loss_ref[0, 0] += jnp.sum(m)

    @pl.when(i == NBLK - 1)
    def _finish():
        loss_ref[0, 0] = loss_ref[0, 0] * ((1.0 + BETA) / (ROWS * D))


_tc_call = pl.pallas_call(
    _tc_body,
    grid=(NBLK,),
    in_specs=[
        pl.BlockSpec((BLK, D), lambda i: (i, 0)),
        pl.BlockSpec((1, K), lambda i: (0, 0)),
        pl.BlockSpec((D, K), lambda i: (0, 0)),
    ],
    out_specs=[
        pl.BlockSpec((1, 1, BLK), lambda i: (i, 0, 0)),
        pl.BlockSpec(memory_space=pltpu.SMEM),
    ],
    out_shape=[
        jax.ShapeDtypeStruct((NBLK, 1, BLK), jnp.int32),
        jax.ShapeDtypeStruct((1, 1), jnp.float32),
    ],
)


_NC, _NS = 2, 16
_NW = _NC * _NS
BPW = ROWS // _NW
CHUNK = 128
NCH = BPW // CHUNK

@functools.cache
def _sc_gather_call():
    mesh = plsc.VectorSubcoreMesh(
        core_axis_name="c", subcore_axis_name="s")

    @functools.partial(
        pl.kernel,
        out_type=jax.ShapeDtypeStruct((ROWS, D), jnp.float32),
        mesh=mesh,
        scratch_types=[
            [pltpu.VMEM((CHUNK,), jnp.int32) for _ in range(NCH)],
            [pltpu.VMEM((CHUNK, D), jnp.float32) for _ in range(NCH)],
            pltpu.SemaphoreType.DMA,
        ],
        compiler_params=pltpu.CompilerParams(use_tc_tiling_on_sc=False),
    )
    def _sc_gather(emb_hbm, idx_hbm, out_hbm, idx_bufs, row_bufs, sem):
        wid = lax.axis_index("s") * _NC + lax.axis_index("c")
        base = wid * BPW
        for j in range(NCH):
            pltpu.sync_copy(
                idx_hbm.at[pl.ds(base + j * CHUNK, CHUNK)], idx_bufs[j])
            pltpu.async_copy(
                emb_hbm.at[idx_bufs[j]], row_bufs[j], sem).wait()
            pltpu.sync_copy(
                row_bufs[j], out_hbm.at[pl.ds(base + j * CHUNK, CHUNK)])

    return _sc_gather


def kernel(z, emb_w):
    z_flat = z.reshape(-1, D)
    se = jnp.sum(emb_w ** 2, axis=1).reshape(1, K)
    emb_t = emb_w.T
    idx3d, loss2d = _tc_call(z_flat, se, emb_t)
    idx = idx3d.reshape(ROWS)
    zq = _sc_gather_call()(emb_w, idx)
    return zq.reshape(z.shape), loss2d[0, 0]


# speedup vs baseline: 1.0412x; 1.0112x over previous
import functools

import jax
import jax.numpy as jnp
from jax import lax
from jax.experimental import pallas as pl
from jax.experimental.pallas import tpu as pltpu
from jax.experimental.pallas import tpu_sc as plsc

D = 64
K = 1024
ROWS = 16 * 1024
BLK = 1024
NBLK = ROWS // BLK
BETA = 0.25


def _tc_body(z_ref, se_ref, emb_t_ref, idx_ref, loss_ref):
    i = pl.program_id(0)
    z = z_ref[0]
    sz = jnp.sum(z * z, axis=1, keepdims=True)
    mm = jnp.dot(z, emb_t_ref[...], preferred_element_type=jnp.float32)
    d = (sz + se_ref[...]) - 2.0 * mm
    m = jnp.min(d, axis=1, keepdims=True)
    iota = lax.broadcasted_iota(jnp.int32, (BLK, K), 1)
    idx = jnp.min(jnp.where(d == m, iota, K), axis=1)
    idx_ref[...] = idx.reshape(1, 1, BLK)

    @pl.when(i == 0)
    def _init():
        loss_ref[0, 0] = 0.0

    loss_ref[0, 0] += jnp.sum(m)

    @pl.when(i == NBLK - 1)
    def _finish():
        loss_ref[0, 0] = loss_ref[0, 0] * ((1.0 + BETA) / (ROWS * D))


_tc_call = pl.pallas_call(
    _tc_body,
    grid=(NBLK,),
    in_specs=[
        pl.BlockSpec((1, BLK, D), lambda i: (i, 0, 0)),
        pl.BlockSpec((1, K), lambda i: (0, 0)),
        pl.BlockSpec((D, K), lambda i: (0, 0)),
    ],
    out_specs=[
        pl.BlockSpec((1, 1, BLK), lambda i: (i, 0, 0)),
        pl.BlockSpec(memory_space=pltpu.SMEM),
    ],
    out_shape=[
        jax.ShapeDtypeStruct((NBLK, 1, BLK), jnp.int32),
        jax.ShapeDtypeStruct((1, 1), jnp.float32),
    ],
)


_NC, _NS = 2, 16
_NW = _NC * _NS
BPW = ROWS // _NW
CHUNK = 128
NCH = BPW // CHUNK

@functools.cache
def _sc_gather_call():
    mesh = plsc.VectorSubcoreMesh(
        core_axis_name="c", subcore_axis_name="s")

    @functools.partial(
        pl.kernel,
        out_type=jax.ShapeDtypeStruct((ROWS, D), jnp.float32),
        mesh=mesh,
        scratch_types=[
            [pltpu.VMEM((CHUNK,), jnp.int32) for _ in range(NCH)],
            [pltpu.VMEM((CHUNK, D), jnp.float32) for _ in range(NCH)],
            pltpu.SemaphoreType.DMA,
        ],
        compiler_params=pltpu.CompilerParams(use_tc_tiling_on_sc=False),
    )
    def _sc_gather(emb_hbm, idx_hbm, out_hbm, idx_bufs, row_bufs, sem):
        wid = lax.axis_index("s") * _NC + lax.axis_index("c")
        base = wid * BPW
        for j in range(NCH):
            pltpu.sync_copy(
                idx_hbm.at[pl.ds(base + j * CHUNK, CHUNK)], idx_bufs[j])
        copies = [
            pltpu.async_copy(emb_hbm.at[idx_bufs[j]], row_bufs[j], sem)
            for j in range(NCH)
        ]
        for j in range(NCH):
            copies[j].wait()
            pltpu.sync_copy(
                row_bufs[j], out_hbm.at[pl.ds(base + j * CHUNK, CHUNK)])

    return _sc_gather


def kernel(z, emb_w):
    se = jnp.sum(emb_w ** 2, axis=1).reshape(1, K)
    emb_t = emb_w.T
    idx3d, loss2d = _tc_call(z, se, emb_t)
    idx = idx3d.reshape(ROWS)
    zq = _sc_gather_call()(emb_w, idx)
    return zq.reshape(z.shape), loss2d[0, 0]


# speedup vs baseline: 1.1761x; 1.1296x over previous
import functools

import jax
import jax.numpy as jnp
from jax import lax
from jax.experimental import pallas as pl
from jax.experimental.pallas import tpu as pltpu
from jax.experimental.pallas import tpu_sc as plsc

D = 64
K = 1024
ROWS = 16 * 1024
BLK = 1024
SUB = 128
NSUB = BLK // SUB
NBLK = ROWS // BLK
BETA = 0.25


GRP = 128
NGRP = K // GRP


def _tc_body(z_ref, se_ref, emb_m2t_ref, idx_ref, loss_ref):
    i = pl.program_id(0)
    z = z_ref[0]
    sz = jnp.sum(z * z, axis=1, keepdims=True)

    lane = lax.broadcasted_iota(jnp.int32, (SUB, GRP), 1)
    loss_part = None
    for r in range(NSUB):
        zr = z[r * SUB:(r + 1) * SUB]
        szr = sz[r * SUB:(r + 1) * SUB]
        val = None
        idx = None
        for c in range(NGRP):
            mm2 = jnp.dot(zr, emb_m2t_ref[:, c * GRP:(c + 1) * GRP],
                          preferred_element_type=jnp.float32)
            d_c = (szr + se_ref[:, c * GRP:(c + 1) * GRP]) + mm2
            if c == 0:
                val, idx = d_c, lane
            else:
                take = d_c < val
                val = jnp.where(take, d_c, val)
                idx = jnp.where(take, lane + c * GRP, idx)
        m = jnp.min(val, axis=1, keepdims=True)
        jmin = jnp.min(jnp.where(val == m, idx, K), axis=1)
        idx_ref[0, 0, r * SUB:(r + 1) * SUB] = jmin
        part = jnp.sum(m)
        loss_part = part if loss_part is None else loss_part + part

    @pl.when(i == 0)
    def _init():
        loss_ref[0, 0] = 0.0

    loss_ref[0, 0] += loss_part

    @pl.when(i == NBLK - 1)
    def _finish():
        loss_ref[0, 0] = loss_ref[0, 0] * ((1.0 + BETA) / (ROWS * D))


_tc_call = pl.pallas_call(
    _tc_body,
    grid=(NBLK,),
    in_specs=[
        pl.BlockSpec((1, BLK, D), lambda i: (i, 0, 0)),
        pl.BlockSpec((1, K), lambda i: (0, 0)),
        pl.BlockSpec((D, K), lambda i: (0, 0)),
    ],
    out_specs=[
        pl.BlockSpec((1, 1, BLK), lambda i: (i, 0, 0)),
        pl.BlockSpec(memory_space=pltpu.SMEM),
    ],
    out_shape=[
        jax.ShapeDtypeStruct((NBLK, 1, BLK), jnp.int32),
        jax.ShapeDtypeStruct((1, 1), jnp.float32),
    ],
)


_NC, _NS = 2, 16
_NW = _NC * _NS
BPW = ROWS // _NW
CHUNK = 128
NCH = BPW // CHUNK

@functools.cache
def _sc_gather_call():
    mesh = plsc.VectorSubcoreMesh(
        core_axis_name="c", subcore_axis_name="s")

    @functools.partial(
        pl.kernel,
        out_type=jax.ShapeDtypeStruct((ROWS, D), jnp.float32),
        mesh=mesh,
        scratch_types=[
            [pltpu.VMEM((CHUNK,), jnp.int32) for _ in range(NCH)],
            [pltpu.VMEM((CHUNK, D), jnp.float32) for _ in range(NCH)],
            pltpu.SemaphoreType.DMA,
        ],
        compiler_params=pltpu.CompilerParams(use_tc_tiling_on_sc=False),
    )
    def _sc_gather(emb_hbm, idx_hbm, out_hbm, idx_bufs, row_bufs, sem):
        wid = lax.axis_index("s") * _NC + lax.axis_index("c")
        base = wid * BPW
        for j in range(NCH):
            pltpu.sync_copy(
                idx_hbm.at[pl.ds(base + j * CHUNK, CHUNK)], idx_bufs[j])
        copies = [
            pltpu.async_copy(emb_hbm.at[idx_bufs[j]], row_bufs[j], sem)
            for j in range(NCH)
        ]
        for j in range(NCH):
            copies[j].wait()
            pltpu.sync_copy(
                row_bufs[j], out_hbm.at[pl.ds(base + j * CHUNK, CHUNK)])

    return _sc_gather


def kernel(z, emb_w):
    se = jnp.sum(emb_w ** 2, axis=1).reshape(1, K)
    emb_m2t = emb_w.T * -2.0
    idx3d, loss2d = _tc_call(z, se, emb_m2t)
    idx = idx3d.reshape(ROWS)
    zq = _sc_gather_call()(emb_w, idx)
    return zq.reshape(z.shape), loss2d[0, 0]
